# baseline (speedup 1.0000x reference)
import jax
import jax.numpy as jnp
from jax import lax
from jax.experimental import pallas as pl
from jax.experimental.pallas import tpu as pltpu

_B, _L, _F, _R = 32, 64, 64, 256
_BB = 4


def _body(ws_ref, wv_ref, x_ref, m_ref, rf_ref, rr_ref,
          car_ref, cdr_ref, cons_ref, max_ref):
    g = pl.program_id(0)

    for i in range(_BB):
        b = g * _BB + i

        def step(l, accs):
            a1, a2 = accs
            xl = x_ref[i, l]
            return (a1 + ws_ref[b, 0, l] * xl, a2 + ws_ref[b, 1, l] * xl)

        z = jnp.zeros((_F, _R), jnp.float32)
        a1, a2 = lax.fori_loop(0, _L, step, (z, z))
        car_ref[i] = jnp.dot(a1, m_ref[0], preferred_element_type=jnp.float32)
        cdr_ref[i] = jnp.dot(a2, m_ref[1], preferred_element_type=jnp.float32)
        cons_ref[i] = (
            jnp.dot(a1, m_ref[2], preferred_element_type=jnp.float32)
            + jnp.dot(a2, m_ref[3], preferred_element_type=jnp.float32)
            + rf_ref[i] * rr_ref[...])

    @pl.when(g == 0)
    def _():
        max_ref[...] = jnp.max(wv_ref[...], axis=-1)


def kernel(x, arg1_weight, arg2_weight, root_filler, D_l, D_r, E_l, E_r, root_role):
    B, L, F, R = _B, _L, _F, _R
    W = jnp.stack([arg1_weight, arg2_weight], axis=1)
    mats = jnp.stack([D_l.T, D_r.T, E_l.T, E_r.T], axis=0)
    rf = root_filler.reshape(B, F, 1)
    rr = root_role.reshape(1, R)
    car, cdr, cons, maxes = pl.pallas_call(
        _body,
        grid=(B // _BB,),
        in_specs=[
            pl.BlockSpec(memory_space=pltpu.SMEM),
            pl.BlockSpec((B, 2, L), lambda g: (0, 0, 0)),
            pl.BlockSpec((_BB, L, F, R), lambda g: (g, 0, 0, 0)),
            pl.BlockSpec((4, R, R), lambda g: (0, 0, 0)),
            pl.BlockSpec((_BB, F, 1), lambda g: (g, 0, 0)),
            pl.BlockSpec((1, R), lambda g: (0, 0)),
        ],
        out_specs=[
            pl.BlockSpec((_BB, F, R), lambda g: (g, 0, 0)),
            pl.BlockSpec((_BB, F, R), lambda g: (g, 0, 0)),
            pl.BlockSpec((_BB, F, R), lambda g: (g, 0, 0)),
            pl.BlockSpec((B, 2), lambda g: (0, 0)),
        ],
        out_shape=[
            jax.ShapeDtypeStruct((B, F, R), jnp.float32),
            jax.ShapeDtypeStruct((B, F, R), jnp.float32),
            jax.ShapeDtypeStruct((B, F, R), jnp.float32),
            jax.ShapeDtypeStruct((B, 2), jnp.float32),
        ],
    )(W, W, x, mats, rf, rr)
    return (car, cdr, cons, maxes[:, 0], maxes[:, 1])
